# baseline (speedup 1.0000x reference)
import jax
import jax.numpy as jnp
from jax.experimental import pallas as pl
from jax.experimental.pallas import tpu as pltpu

E = 8
D = 1024
R = 16
DQ = 1024
DV = 1024
SCALE = 32.0 / 16.0
ER = E * R
T = 2048
CHUNK = 512
NC = T // CHUNK


def _router_lora_kernel(h_hbm, wrt_hbm, qa_hbm, qb_hbm, va_hbm, vb_hbm,
                        q_hbm, v_hbm,
                        h_v, wrt_v, qa_v, qb_v, va_v, vb_v, q_v, v_v,
                        h_sem, w_sem, out_sem):
    wrt_cp = pltpu.make_async_copy(wrt_hbm, wrt_v, w_sem.at[0])
    qa_cp = pltpu.make_async_copy(qa_hbm, qa_v, w_sem.at[1])
    va_cp = pltpu.make_async_copy(va_hbm, va_v, w_sem.at[2])
    qb_cp = pltpu.make_async_copy(qb_hbm, qb_v, w_sem.at[3])
    vb_cp = pltpu.make_async_copy(vb_hbm, vb_v, w_sem.at[4])
    h_cps = [pltpu.make_async_copy(h_hbm.at[pl.ds(c * CHUNK, CHUNK), :],
                                   h_v.at[c], h_sem.at[c])
             for c in range(NC)]
    wrt_cp.start()
    h_cps[0].start()
    qa_cp.start()
    va_cp.start()
    qb_cp.start()
    vb_cp.start()
    for c in range(1, NC):
        h_cps[c].start()

    wrt_cp.wait()
    qa_cp.wait()
    va_cp.wait()
    qb_cp.wait()
    vb_cp.wait()
    out_cps = []
    for c in range(NC):
        h_cps[c].wait()
        h = h_v[c]
        logits = jnp.dot(h, wrt_v[...], preferred_element_type=jnp.float32)
        m = jnp.max(logits, axis=1, keepdims=True)
        score = 1.0 / jnp.sum(jnp.exp(logits - m), axis=1, keepdims=True)
        idx = jnp.argmax(logits, axis=1)
        col_expert = jax.lax.broadcasted_iota(jnp.int32, (CHUNK, ER), 1) // R
        mask = jnp.where(col_expert == idx[:, None], score * SCALE, 0.0)
        lr_q = jnp.dot(h, qa_v[...], preferred_element_type=jnp.float32) * mask
        q_v[c] = jnp.dot(lr_q, qb_v[...], preferred_element_type=jnp.float32)
        qo = pltpu.make_async_copy(q_v.at[c],
                                   q_hbm.at[pl.ds(c * CHUNK, CHUNK), :],
                                   out_sem.at[2 * c])
        qo.start()
        lr_v = jnp.dot(h, va_v[...], preferred_element_type=jnp.float32) * mask
        v_v[c] = jnp.dot(lr_v, vb_v[...], preferred_element_type=jnp.float32)
        vo = pltpu.make_async_copy(v_v.at[c],
                                   v_hbm.at[pl.ds(c * CHUNK, CHUNK), :],
                                   out_sem.at[2 * c + 1])
        vo.start()
        out_cps.extend([qo, vo])
    for cp in out_cps:
        cp.wait()


def kernel(hidden_states, router_weight, q_lora_a, q_lora_b, v_lora_a, v_lora_b):
    orig_shape = hidden_states.shape[:-1]
    h = hidden_states.reshape(-1, D)
    wrt = router_weight.T
    qa = q_lora_a.transpose(1, 0, 2).reshape(D, ER)
    qb = q_lora_b.reshape(ER, DQ)
    va = v_lora_a.transpose(1, 0, 2).reshape(D, ER)
    vb = v_lora_b.reshape(ER, DV)

    any_spec = pl.BlockSpec(memory_space=pltpu.MemorySpace.HBM)
    q_out, v_out = pl.pallas_call(
        _router_lora_kernel,
        in_specs=[any_spec] * 6,
        out_specs=[any_spec, any_spec],
        out_shape=[
            jax.ShapeDtypeStruct((T, DQ), jnp.float32),
            jax.ShapeDtypeStruct((T, DV), jnp.float32),
        ],
        scratch_shapes=[
            pltpu.VMEM((NC, CHUNK, D), jnp.float32),
            pltpu.VMEM((D, E), jnp.float32),
            pltpu.VMEM((D, ER), jnp.float32),
            pltpu.VMEM((ER, DQ), jnp.float32),
            pltpu.VMEM((D, ER), jnp.float32),
            pltpu.VMEM((ER, DV), jnp.float32),
            pltpu.VMEM((NC, CHUNK, DQ), jnp.float32),
            pltpu.VMEM((NC, CHUNK, DV), jnp.float32),
            pltpu.SemaphoreType.DMA((NC,)),
            pltpu.SemaphoreType.DMA((5,)),
            pltpu.SemaphoreType.DMA((2 * NC,)),
        ],
    )(h, wrt, qa, qb, va, vb)
    return (q_out.reshape(orig_shape + (DQ,)),
            v_out.reshape(orig_shape + (DV,)))
